# baseline (speedup 1.0000x reference)
import functools

import jax
import jax.numpy as jnp
from jax import lax
from jax.experimental import pallas as pl
from jax.experimental.pallas import tpu as pltpu
from jax.experimental.pallas import tpu_sc as plsc

_CB = 128
_E = 32
_ROWS = 131072
_BATCH = 16384


_BLK_R = 8192


def _argmin_body(cb_ref, lat_ref, idx_ref):
    cb = cb_ref[...]
    z = lat_ref[...]
    d = -2.0 * lax.dot_general(
        z, cb.T, (((1,), (0,)), ((), ())),
        preferred_element_type=jnp.float32)
    d = d + jnp.sum(cb * cb, axis=1)[None, :]
    m = jnp.min(d, axis=-1, keepdims=True)
    ii = lax.broadcasted_iota(jnp.int32, d.shape, 1)
    idx = jnp.min(jnp.where(d == m, ii, _CB), axis=-1, keepdims=True)
    idx_ref[...] = idx


def _tc_argmin(latent_flat, codebook):
    return pl.pallas_call(
        _argmin_body,
        grid=(_ROWS // _BLK_R,),
        in_specs=[
            pl.BlockSpec((_CB, _E), lambda i: (0, 0)),
            pl.BlockSpec((_BLK_R, _E), lambda i: (i, 0)),
        ],
        out_specs=pl.BlockSpec((_BLK_R, 1), lambda i: (i, 0)),
        out_shape=jax.ShapeDtypeStruct((_ROWS, 1), jnp.int32),
    )(codebook, latent_flat)


_NC, _NS = 2, 16
_NW = _NC * _NS
_GL = 128
_NG = _ROWS // _GL // _NW
_K = 8
_NGRP = _NG // _K


def _sc_gather_body(cb_hbm, idx_hbm, out1, out2, idx_v, rows_v, sem):
    wid = lax.axis_index("s") * _NC + lax.axis_index("c")
    base = wid * _NG
    pltpu.sync_copy(idx_hbm.at[pl.ds(base, _NG)], idx_v)
    for g in range(_NGRP):
        handles = [
            pltpu.async_copy(cb_hbm.at[idx_v.at[g * _K + j]], rows_v.at[j], sem)
            for j in range(_K)
        ]
        for h in handles:
            h.wait()
        pltpu.sync_copy(rows_v, out1.at[pl.ds(base + g * _K, _K)])
        pltpu.sync_copy(rows_v, out2.at[pl.ds(base + g * _K, _K)])


@functools.cache
def _sc_gather_kernel():
    return pl.kernel(
        _sc_gather_body,
        out_type=(
            jax.ShapeDtypeStruct((_ROWS // _GL, _GL, _E), jnp.float32),
            jax.ShapeDtypeStruct((_ROWS // _GL, _GL, _E), jnp.float32),
        ),
        mesh=plsc.VectorSubcoreMesh(core_axis_name="c", subcore_axis_name="s"),
        scratch_types=[
            pltpu.VMEM((_NG, _GL), jnp.int32),
            pltpu.VMEM((_K, _GL, _E), jnp.float32),
            pltpu.SemaphoreType.DMA,
        ],
        compiler_params=pltpu.CompilerParams(use_tc_tiling_on_sc=False),
    )


_BLK_B = 512


def _bcast_body(cb_ref, out_ref):
    out_ref[...] = jnp.broadcast_to(cb_ref[...], (_BLK_B, _CB * _E))


def _tc_broadcast(cb_row):
    return pl.pallas_call(
        _bcast_body,
        grid=(_BATCH // _BLK_B,),
        in_specs=[pl.BlockSpec((1, _CB * _E), lambda i: (0, 0))],
        out_specs=pl.BlockSpec((_BLK_B, _CB * _E), lambda i: (i, 0)),
        out_shape=jax.ShapeDtypeStruct((_BATCH, _CB * _E), jnp.float32),
    )(cb_row)


def kernel(latent, codebook):
    latent_flat = latent.reshape(_ROWS, _E)
    idx = _tc_argmin(latent_flat, codebook)
    q1, q2 = _sc_gather_kernel()(codebook, idx.reshape(_ROWS // _GL, _GL))
    cbw = _tc_broadcast(codebook.reshape(1, _CB * _E))
    policy = q1.reshape(latent.shape)
    quantized = q2.reshape(latent.shape)
    return policy, quantized, cbw.reshape(_BATCH, _CB, _E)


# speedup vs baseline: 1.2740x; 1.2740x over previous
import functools

import jax
import jax.numpy as jnp
from jax import lax
from jax.experimental import pallas as pl
from jax.experimental.pallas import tpu as pltpu
from jax.experimental.pallas import tpu_sc as plsc

_CB = 128
_E = 32
_ROWS = 131072
_BATCH = 16384
_ROW_F = _CB * _E


_NC, _NS = 2, 16
_NW = _NC * _NS
_BPW = _BATCH // _NW
_BUF = 16
_NST = _BPW // _BUF
_KD = 8


def _sc_bcast_body(cb_hbm, out, buf, sem):
    wid = lax.axis_index("s") * _NC + lax.axis_index("c")
    base = wid * _BPW
    for j in range(_BUF):
        pltpu.sync_copy(cb_hbm, buf.at[j])
    for g in range(_NST // _KD):
        handles = [
            pltpu.async_copy(
                buf, out.at[pl.ds(base + (g * _KD + t) * _BUF, _BUF)], sem)
            for t in range(_KD)
        ]
        for h in handles:
            h.wait()


@functools.cache
def _sc_bcast_kernel():
    return pl.kernel(
        _sc_bcast_body,
        out_type=jax.ShapeDtypeStruct((_BATCH, _ROW_F), jnp.float32),
        mesh=plsc.VectorSubcoreMesh(core_axis_name="c", subcore_axis_name="s"),
        scratch_types=[
            pltpu.VMEM((_BUF, _ROW_F), jnp.float32),
            pltpu.SemaphoreType.DMA,
        ],
    )


_BLK_R = 8192


def _quant_body(cb_ref, lat_ref, q1_ref, q2_ref):
    cb = cb_ref[...]
    z = lat_ref[...]
    d = -2.0 * lax.dot_general(
        z, cb.T, (((1,), (0,)), ((), ())),
        preferred_element_type=jnp.float32)
    d = d + jnp.sum(cb * cb, axis=1)[None, :]
    m = jnp.min(d, axis=-1, keepdims=True)
    ii = lax.broadcasted_iota(jnp.int32, d.shape, 1)
    idx = jnp.min(jnp.where(d == m, ii, _CB), axis=-1, keepdims=True)
    e = (ii == idx).astype(jnp.float32)
    q = lax.dot_general(e, cb, (((1,), (0,)), ((), ())),
                        preferred_element_type=jnp.float32)
    q1_ref[...] = q
    q2_ref[...] = q


def _tc_quantize(latent_flat, codebook):
    out = jax.ShapeDtypeStruct((_ROWS, _E), jnp.float32)
    return pl.pallas_call(
        _quant_body,
        grid=(_ROWS // _BLK_R,),
        in_specs=[
            pl.BlockSpec((_CB, _E), lambda i: (0, 0)),
            pl.BlockSpec((_BLK_R, _E), lambda i: (i, 0)),
        ],
        out_specs=[
            pl.BlockSpec((_BLK_R, _E), lambda i: (i, 0)),
            pl.BlockSpec((_BLK_R, _E), lambda i: (i, 0)),
        ],
        out_shape=[out, out],
    )(codebook, latent_flat)


def kernel(latent, codebook):
    cbw = _sc_bcast_kernel()(codebook.reshape(_ROW_F))
    q1, q2 = _tc_quantize(latent.reshape(_ROWS, _E), codebook)
    policy = q1.reshape(latent.shape)
    quantized = q2.reshape(latent.shape)
    return policy, quantized, cbw.reshape(_BATCH, _CB, _E)


# speedup vs baseline: 5.6511x; 4.4359x over previous
import functools

import jax
import jax.numpy as jnp
from jax import lax
from jax.experimental import pallas as pl
from jax.experimental.pallas import tpu as pltpu
from jax.experimental.pallas import tpu_sc as plsc

_CB = 128
_E = 32
_L = 8
_BATCH = 16384
_ROW_F = _CB * _E


_NC, _NS = 2, 16
_NW = _NC * _NS
_BPW = _BATCH // _NW
_BUF = 16
_NST = _BPW // _BUF
_KD = 8


def _sc_bcast_body(cbt_hbm, out, buf, sem):
    wid = lax.axis_index("s") * _NC + lax.axis_index("c")
    base = wid * _BPW * _E
    for j in range(_BUF):
        pltpu.sync_copy(cbt_hbm, buf.at[pl.ds(j * _E, _E)])
    for g in range(_NST // _KD):
        handles = [
            pltpu.async_copy(
                buf,
                out.at[pl.ds(base + (g * _KD + t) * _BUF * _E, _BUF * _E)],
                sem)
            for t in range(_KD)
        ]
        for h in handles:
            h.wait()


@functools.cache
def _sc_bcast_kernel():
    return pl.kernel(
        _sc_bcast_body,
        out_type=jax.ShapeDtypeStruct((_BATCH * _E, _CB), jnp.float32),
        mesh=plsc.VectorSubcoreMesh(core_axis_name="c", subcore_axis_name="s"),
        scratch_types=[
            pltpu.VMEM((_BUF * _E, _CB), jnp.float32),
            pltpu.SemaphoreType.DMA,
        ],
    )


_BLK_B = 2048


def _quant_body(cb_ref, lat_ref, q1_ref, q2_ref):
    cb = cb_ref[...]
    cnorm = jnp.sum(cb * cb, axis=1, keepdims=True)
    for l in range(_L):
        z = lat_ref[l]
        d = -2.0 * lax.dot_general(
            cb, z, (((1,), (0,)), ((), ())),
            preferred_element_type=jnp.float32)
        d = d + cnorm
        m = jnp.min(d, axis=0, keepdims=True)
        ii = lax.broadcasted_iota(jnp.int32, d.shape, 0)
        idx = jnp.min(jnp.where(d == m, ii, _CB), axis=0, keepdims=True)
        e = (ii == idx).astype(jnp.float32)
        q = lax.dot_general(cb, e, (((0,), (0,)), ((), ())),
                            preferred_element_type=jnp.float32)
        q1_ref[l] = q
        q2_ref[l] = q


def _tc_quantize(latent_t, codebook):
    out = jax.ShapeDtypeStruct((_L, _E, _BATCH), jnp.float32)
    blk = pl.BlockSpec((_L, _E, _BLK_B), lambda i: (0, 0, i))
    return pl.pallas_call(
        _quant_body,
        grid=(_BATCH // _BLK_B,),
        in_specs=[pl.BlockSpec((_CB, _E), lambda i: (0, 0)), blk],
        out_specs=[blk, blk],
        out_shape=[out, out],
    )(codebook, latent_t)


def kernel(latent, codebook):
    latent_t = jnp.transpose(latent, (1, 2, 0))
    q1, q2 = _tc_quantize(latent_t, codebook)
    policy = jnp.transpose(q1, (2, 0, 1))
    quantized = jnp.transpose(q2, (2, 0, 1))
    cbw = _sc_bcast_kernel()(codebook.T)
    codebook_weight = jnp.swapaxes(cbw.reshape(_BATCH, _E, _CB), 1, 2)
    return policy, quantized, codebook_weight
